# baseline (speedup 1.0000x reference)
import functools

import jax
import jax.numpy as jnp
from jax import lax
from jax.experimental import pallas as pl
from jax.experimental.pallas import tpu as pltpu
from jax.experimental.pallas import tpu_sc as plsc

D = 64
NBUF = 4
LA = 2


@jax.jit
def _emb_lookup(token_ids, weight):
    B, T = token_ids.shape
    info = plsc.get_sparse_core_info()
    nw = info.num_cores * info.num_subcores
    rows_per_w = B // nw
    mesh = plsc.VectorSubcoreMesh(core_axis_name="c", subcore_axis_name="s")

    @functools.partial(
        pl.kernel,
        mesh=mesh,
        compiler_params=pltpu.CompilerParams(use_tc_tiling_on_sc=False),
        out_type=jax.ShapeDtypeStruct((B, T, D), jnp.float32),
        scratch_types=[
            pltpu.VMEM((rows_per_w, T), jnp.int32),
            *[pltpu.VMEM((T, D), jnp.float32) for _ in range(NBUF)],
            *[pltpu.SemaphoreType.DMA for _ in range(2 * NBUF)],
        ],
    )
    def emb(ids_hbm, table_hbm, out_hbm, idx_all, *bufs_and_sems):
        bufs = bufs_and_sems[:NBUF]
        gsem = bufs_and_sems[NBUF:2 * NBUF]
        wsem = bufs_and_sems[2 * NBUF:]
        wid = lax.axis_index("s") * info.num_cores + lax.axis_index("c")
        base = wid * rows_per_w
        n_groups = rows_per_w // NBUF

        def fire_gather(b, r):
            pltpu.async_copy(table_hbm.at[idx_all.at[r]], bufs[b], gsem[b])

        def drain_gather(b):
            pltpu.make_async_copy(table_hbm.at[idx_all.at[0]], bufs[b],
                                  gsem[b]).wait()

        def fire_write(b, r):
            pltpu.async_copy(bufs[b], out_hbm.at[base + r], wsem[b])

        def drain_write(b):
            pltpu.make_async_copy(bufs[b], out_hbm.at[base], wsem[b]).wait()

        pltpu.sync_copy(ids_hbm.at[pl.ds(base, rows_per_w)], idx_all)

        for r in range(LA):
            fire_gather(r % NBUF, r)

        def body(g, carry):
            for b in range(NBUF):
                r = g * NBUF + b
                bn = (b + LA) % NBUF
                if b + LA >= NBUF:
                    drain_write(bn)
                else:

                    @pl.when(g >= 1)
                    def _():
                        drain_write(bn)

                if b + LA < NBUF:
                    fire_gather(bn, r + LA)
                else:

                    @pl.when(g < n_groups - 1)
                    def _():
                        fire_gather(bn, r + LA)

                drain_gather(b)
                fire_write(b, r)
            return carry

        lax.fori_loop(0, n_groups, body, 0)

        for r in range(rows_per_w - LA, rows_per_w):
            drain_write(r % NBUF)

    return emb(token_ids, weight)


def kernel(token_ids, weight):
    return _emb_lookup(token_ids.astype(jnp.int32), weight)


# speedup vs baseline: 1.0013x; 1.0013x over previous
import functools

import jax
import jax.numpy as jnp
from jax import lax
from jax.experimental import pallas as pl
from jax.experimental.pallas import tpu as pltpu
from jax.experimental.pallas import tpu_sc as plsc

D = 64
NBUF = 4
LA = 2


@jax.jit
def _emb_lookup(ids_tb, weight):
    T, B = ids_tb.shape
    info = plsc.get_sparse_core_info()
    nw = info.num_cores * info.num_subcores
    cols_per_w = B // nw
    mesh = plsc.VectorSubcoreMesh(core_axis_name="c", subcore_axis_name="s")

    @functools.partial(
        pl.kernel,
        mesh=mesh,
        compiler_params=pltpu.CompilerParams(use_tc_tiling_on_sc=False),
        out_type=jax.ShapeDtypeStruct((B, T, D), jnp.float32),
        scratch_types=[
            pltpu.VMEM((T, cols_per_w), jnp.int32),
            *[pltpu.VMEM((cols_per_w, D), jnp.float32) for _ in range(NBUF)],
            *[pltpu.SemaphoreType.DMA for _ in range(2 * NBUF)],
        ],
    )
    def emb(ids_hbm, table_hbm, out_hbm, idx_all, *bufs_and_sems):
        bufs = bufs_and_sems[:NBUF]
        gsem = bufs_and_sems[NBUF:2 * NBUF]
        wsem = bufs_and_sems[2 * NBUF:]
        wid = lax.axis_index("s") * info.num_cores + lax.axis_index("c")
        col0 = wid * cols_per_w
        n_groups = T // NBUF

        def fire_gather(b, t):
            pltpu.async_copy(table_hbm.at[idx_all.at[t]], bufs[b], gsem[b])

        def drain_gather(b):
            pltpu.make_async_copy(table_hbm.at[idx_all.at[0]], bufs[b],
                                  gsem[b]).wait()

        def fire_write(b, t):
            pltpu.async_copy(bufs[b],
                             out_hbm.at[pl.ds(col0, cols_per_w), t], wsem[b])

        def drain_write(b):
            pltpu.make_async_copy(bufs[b],
                                  out_hbm.at[pl.ds(col0, cols_per_w), 0],
                                  wsem[b]).wait()

        pltpu.sync_copy(ids_hbm.at[pl.ds(0, T), pl.ds(col0, cols_per_w)],
                        idx_all)

        for t in range(LA):
            fire_gather(t % NBUF, t)

        def body(g, carry):
            for b in range(NBUF):
                t = g * NBUF + b
                bn = (b + LA) % NBUF
                if b + LA >= NBUF:
                    drain_write(bn)
                else:

                    @pl.when(g >= 1)
                    def _():
                        drain_write(bn)

                if b + LA < NBUF:
                    fire_gather(bn, t + LA)
                else:

                    @pl.when(g < n_groups - 1)
                    def _():
                        fire_gather(bn, t + LA)

                drain_gather(b)
                fire_write(b, t)
            return carry

        lax.fori_loop(0, n_groups, body, 0)

        for t in range(T - LA, T):
            drain_write(t % NBUF)

    return emb(ids_tb, weight)


def kernel(token_ids, weight):
    return _emb_lookup(token_ids.T.astype(jnp.int32), weight)
